# baseline (speedup 1.0000x reference)
import functools

import jax
import jax.numpy as jnp
from jax import lax
from jax.experimental import pallas as pl
from jax.experimental.pallas import tpu as pltpu
from jax.experimental.pallas import tpu_sc as plsc

N = 10000
E = 320000
D = 128
DE = 16
H = 256
M = 128

NC = 2
NS = 16
NW = NC * NS

_K = 5
_EK = E // _K
_RW = (2 * _EK) // NW
_CH = 80
_NG = _RW // _CH

_NPAD = 10240
_N_PER_S = _NPAD // NS


_GB = 4


def _gather_body(table_hbm, idx_hbm, out_hbm,
                 idx_res, r0, r1, r2, r3,
                 g0, g1, g2, g3, t0, t1, t2, t3):
    c = lax.axis_index("c")
    s = lax.axis_index("s")
    wid = s * NC + c
    rows = (r0, r1, r2, r3)
    gsem = (g0, g1, g2, g3)
    tsem = (t0, t1, t2, t3)

    orow = (wid % NS) * _RW
    ocol = pl.multiple_of(jnp.where(wid < NS, 0, D), D)

    pltpu.sync_copy(idx_hbm.at[wid], idx_res)

    def fetch(g, b):
        pltpu.async_copy(table_hbm.at[idx_res.at[g]], rows[b], gsem[b])

    def fetch_wait(g, b):
        pltpu.make_async_copy(
            table_hbm.at[idx_res.at[g]], rows[b], gsem[b]).wait()

    def store(g, b):
        pltpu.async_copy(
            rows[b], out_hbm.at[pl.ds(orow + g * _CH, _CH), pl.ds(ocol, D)],
            tsem[b])

    def store_wait(g, b):
        pltpu.make_async_copy(
            rows[b], out_hbm.at[pl.ds(orow + g * _CH, _CH), pl.ds(ocol, D)],
            tsem[b]).wait()

    def step(q, _):
        for j in range(_GB):
            g = q * _GB + j

            @pl.when(g >= _GB)
            def _():
                store_wait(g - _GB, j)

            fetch(g, j)
            b2 = (j - 2) % _GB

            @pl.when(g >= 2)
            def _():
                fetch_wait(g - 2, b2)
                store(g - 2, b2)
        return 0

    nq = _NG // _GB
    lax.fori_loop(0, nq, step, 0)

    for g in (nq * _GB, nq * _GB + 1):
        b = g % _GB
        store_wait(g - _GB, b)
        fetch(g, b)
    for g in range(_NG - _GB, _NG):
        fetch_wait(g, g % _GB)
        store(g, g % _GB)
    for g in range(_NG - _GB, _NG):
        store_wait(g, g % _GB)


def _sc_gather(table, idx):
    mesh = plsc.VectorSubcoreMesh(core_axis_name="c", subcore_axis_name="s")
    f = functools.partial(
        pl.kernel,
        mesh=mesh,
        out_type=jax.ShapeDtypeStruct((_EK, 2 * D), jnp.float32),
        scratch_types=[pltpu.VMEM((_NG, _CH), jnp.int32)]
                      + [pltpu.VMEM((_CH, D), jnp.float32)] * _GB
                      + [pltpu.SemaphoreType.DMA] * (2 * _GB),
    )(_gather_body)
    return f(table, idx)


_SB = 3


def _scatter_body(msg_hbm, sidx_hbm, z_hbm, out_hbm,
                  idx_res, m0, m1, m2, l0, l1, l2, a0, a1, a2, acc_sh):
    c = lax.axis_index("c")
    s = lax.axis_index("s")
    wid = s * NC + c
    base0 = wid * _RW
    msgs = (m0, m1, m2)
    lsem = (l0, l1, l2)
    asem = (a0, a1, a2)

    pltpu.sync_copy(
        z_hbm.at[pl.ds(c * _NPAD + s * _N_PER_S, _N_PER_S)],
        acc_sh.at[pl.ds(s * _N_PER_S, _N_PER_S)])
    pltpu.sync_copy(sidx_hbm.at[wid], idx_res)
    plsc.subcore_barrier()

    def load(g, b):
        pltpu.async_copy(
            msg_hbm.at[pl.ds(base0 + g * _CH, _CH)], msgs[b], lsem[b])

    def load_wait(g, b):
        pltpu.make_async_copy(
            msg_hbm.at[pl.ds(base0 + g * _CH, _CH)], msgs[b], lsem[b]).wait()

    def scat(g, b):
        pltpu.async_copy(msgs[b], acc_sh.at[idx_res.at[g]], asem[b], add=True)

    def scat_wait(g, b):
        pltpu.make_async_copy(msgs[b], acc_sh.at[idx_res.at[g]], asem[b]).wait()

    def step(q, _):
        for j in range(_SB):
            g = q * _SB + j

            @pl.when(g >= _SB)
            def _():
                scat_wait(g - _SB, j)

            load(g, j)
            b2 = (j - 1) % _SB

            @pl.when(g >= 1)
            def _():
                load_wait(g - 1, b2)
                scat(g - 1, b2)
        return 0

    nq = _NG // _SB
    lax.fori_loop(0, nq, step, 0)

    for g in (nq * _SB, nq * _SB + 1):
        b = g % _SB
        scat_wait(g - _SB, b)
        load(g, b)
        load_wait(g - 1, (g - 1) % _SB)
        scat(g - 1, (g - 1) % _SB)
    load_wait(_NG - 1, (_NG - 1) % _SB)
    scat(_NG - 1, (_NG - 1) % _SB)
    for g in range(_NG - _SB, _NG):
        scat_wait(g, g % _SB)

    plsc.subcore_barrier()
    pltpu.sync_copy(
        acc_sh.at[pl.ds(s * _N_PER_S, _N_PER_S)],
        out_hbm.at[pl.ds(c * _NPAD + s * _N_PER_S, _N_PER_S)],
    )


def _sc_scatter(msg, sidx, init):
    mesh = plsc.VectorSubcoreMesh(core_axis_name="c", subcore_axis_name="s")
    f = functools.partial(
        pl.kernel,
        mesh=mesh,
        out_type=jax.ShapeDtypeStruct((NC * _NPAD, M), jnp.float32),
        scratch_types=[pltpu.VMEM((_NG, _CH), jnp.int32)]
                      + [pltpu.VMEM((_CH, M), jnp.float32)] * _SB
                      + [pltpu.SemaphoreType.DMA] * (2 * _SB)
                      + [pltpu.VMEM_SHARED((_NPAD, M), jnp.float32)],
    )(_scatter_body)
    return f(msg, sidx, init)


_BE = 6400


_TDN = (((0,), (0,)), ((), ()))


def _msg_body(ft, eft, u1m, wm1e, bm1, wm2, bm2,
              u1r, wr1e, br1, wr2, br2, out):
    x = ft[...].astype(jnp.bfloat16)
    e = eft[...].astype(jnp.bfloat16)
    hf = jnp.maximum(
        jnp.dot(x, u1m[...], preferred_element_type=jnp.float32)
        + lax.dot_general(e, wm1e[...], _TDN,
                          preferred_element_type=jnp.float32)
        + bm1[...], 0.0).astype(jnp.bfloat16)
    out[0] = jnp.dot(hf, wm2[...], preferred_element_type=jnp.float32) + bm2[...]
    hr = jnp.maximum(
        jnp.dot(x, u1r[...], preferred_element_type=jnp.float32)
        + lax.dot_general(e, wr1e[...], _TDN,
                          preferred_element_type=jnp.float32)
        + br1[...], 0.0).astype(jnp.bfloat16)
    out[1] = jnp.dot(hr, wr2[...], preferred_element_type=jnp.float32) + br2[...]


def _tc_messages(ft, eft, weights, k):
    grid = (_EK // _BE,)
    full = lambda a: pl.BlockSpec(a.shape, lambda i: (0,) * a.ndim)
    off = k * (_EK // _BE)
    return pl.pallas_call(
        _msg_body,
        grid=grid,
        in_specs=[pl.BlockSpec((_BE, 2 * D), lambda i: (i, 0)),
                  pl.BlockSpec((DE, _BE), lambda i: (0, i + off))]
                 + [full(w) for w in weights],
        out_specs=pl.BlockSpec((2, _BE, M), lambda i: (0, i, 0)),
        out_shape=jax.ShapeDtypeStruct((2, _EK, M), jnp.float32),
    )(ft, eft, *weights)


_BN = 1000


def _update_body(ns, p, wa, wb, bn, out):
    x = ns[...]
    agg = p[0] + p[1]
    out[...] = (x + bn[...]
                + jnp.dot(x, wa[...], preferred_element_type=jnp.float32)
                + jnp.dot(agg, wb[...], preferred_element_type=jnp.float32))


def _tc_update(ns, parts, Wn, bn):
    grid = (N // _BN,)
    full = lambda a: pl.BlockSpec(a.shape, lambda i: (0,) * a.ndim)
    wa, wb, bnr = Wn[:D], Wn[D:], bn.reshape(1, D)
    return pl.pallas_call(
        _update_body,
        grid=grid,
        in_specs=[pl.BlockSpec((_BN, D), lambda i: (i, 0)),
                  pl.BlockSpec((NC, _BN, M), lambda i: (0, i, 0)),
                  full(wa), full(wb), full(bnr)],
        out_specs=pl.BlockSpec((_BN, D), lambda i: (i, 0)),
        out_shape=jax.ShapeDtypeStruct((N, D), jnp.float32),
    )(ns, parts, wa, wb, bnr)


def kernel(node_states, from_idx, to_idx, edge_features,
           Wm1, bm1, Wm2, bm2, Wr1, br1, Wr2, br2, Wn, bn):
    b16 = lambda a: a.astype(jnp.bfloat16)
    u1r = jnp.concatenate([Wr1[D:2 * D], Wr1[:D]], axis=0)
    weights = [b16(Wm1[:2 * D]), b16(Wm1[2 * D:]), bm1.reshape(1, H),
               b16(Wm2), bm2.reshape(1, M),
               b16(u1r), b16(Wr1[2 * D:]), br1.reshape(1, H),
               b16(Wr2), br2.reshape(1, M)]
    part = jnp.zeros((NC * _NPAD, M), jnp.float32)
    eft = edge_features.T

    for k in range(_K):
        fr = lax.dynamic_slice_in_dim(from_idx, k * _EK, _EK)
        to = lax.dynamic_slice_in_dim(to_idx, k * _EK, _EK)
        gidx = jnp.concatenate([fr, to]).reshape(NW, _NG, _CH)
        sidx = jnp.concatenate([to, fr]).reshape(NW, _NG, _CH)
        ft = _sc_gather(node_states, gidx)
        msg = _tc_messages(ft, eft, weights, k)
        part = _sc_scatter(msg.reshape(2 * _EK, M), sidx, part)

    return _tc_update(node_states, part.reshape(NC, _NPAD, M), Wn, bn)
